# baseline (speedup 1.0000x reference)
import functools

import jax
import jax.numpy as jnp
from jax import lax
from jax.experimental import pallas as pl
from jax.experimental.pallas import tpu as pltpu
from jax.experimental.pallas import tpu_sc as plsc

N_TOK = 8192
N_CB = 8192
D = 256

BM = 512
BW = 1024
NF = N_CB // BW
NI = N_TOK // BM


def _dist_argmin_body(x_ref, ct_ref, ids_ref, d2_ref, c2_ref):
    i = pl.program_id(0)

    @pl.when(i == 0)
    def _cache_c2():
        ct = ct_ref[...]
        c2_ref[...] = jnp.sum(ct * ct, axis=0, keepdims=True)

    x = x_ref[...]
    xm2 = x * -2.0

    vals = [
        c2_ref[:, jj * BW:(jj + 1) * BW]
        + lax.dot_general(xm2, ct_ref[:, jj * BW:(jj + 1) * BW],
                          (((1,), (0,)), ((), ())),
                          preferred_element_type=jnp.float32)
        for jj in range(NF)
    ]
    idxs = [jnp.full((BM, BW), jj, jnp.int32) for jj in range(NF)]
    while len(vals) > 1:
        nv, ni_ = [], []
        for k in range(0, len(vals), 2):
            va, vb = vals[k], vals[k + 1]
            ia, ib = idxs[k], idxs[k + 1]
            take_b = vb < va
            nv.append(jnp.minimum(va, vb))
            ni_.append(jnp.where(take_b, ib, ia))
        vals, idxs = nv, ni_
    a, ja = vals[0], idxs[0]

    m = jnp.min(a, axis=1, keepdims=True)
    col = ja * BW + lax.broadcasted_iota(jnp.int32, (BM, BW), 1)
    ids_ref[...] = jnp.min(jnp.where(a == m, col, N_CB), axis=1,
                           keepdims=True)
    x2 = jnp.sum(x * x, axis=1, keepdims=True)
    d2_ref[...] = jnp.maximum(x2 + m, 0.0)


_dist_argmin = pl.pallas_call(
    _dist_argmin_body,
    grid=(NI,),
    in_specs=[
        pl.BlockSpec((BM, D), lambda i: (i, 0)),
        pl.BlockSpec((D, N_CB), lambda i: (0, 0)),
    ],
    out_specs=[
        pl.BlockSpec((BM, 1), lambda i: (i, 0)),
        pl.BlockSpec((BM, 1), lambda i: (i, 0)),
    ],
    out_shape=[
        jax.ShapeDtypeStruct((N_TOK, 1), jnp.int32),
        jax.ShapeDtypeStruct((N_TOK, 1), jnp.float32),
    ],
    scratch_shapes=[
        pltpu.VMEM((1, N_CB), jnp.float32),
    ],
    compiler_params=pltpu.CompilerParams(
        dimension_semantics=("arbitrary",),
    ),
)

_NC = 2
_NS = 16
_NW = _NC * _NS
_BPW = N_TOK // _NW
_CHUNK = 128
_NCH = _BPW // _CHUNK


def _sc_gather_body(table_hbm, idx_hbm, out_hbm, idx_v, rows_v, sem):
    wid = lax.axis_index("s") * _NC + lax.axis_index("c")
    base = wid * _BPW
    pltpu.sync_copy(idx_hbm.at[wid], idx_v)
    copies = []
    for k in range(_NCH):
        copies.append(pltpu.async_copy(
            table_hbm.at[idx_v.at[k]],
            rows_v.at[pl.ds(k * _CHUNK, _CHUNK)],
            sem,
        ))
    for cp in copies:
        cp.wait()
    pltpu.sync_copy(rows_v, out_hbm.at[pl.ds(base, _BPW)])


@functools.cache
def _sc_gather():
    return pl.kernel(
        _sc_gather_body,
        mesh=plsc.VectorSubcoreMesh(core_axis_name="c", subcore_axis_name="s"),
        out_type=jax.ShapeDtypeStruct((N_TOK, D), jnp.float32),
        scratch_types=[
            pltpu.VMEM((_NCH, _CHUNK), jnp.int32),
            pltpu.VMEM((_BPW, D), jnp.float32),
            pltpu.SemaphoreType.DMA,
        ],
    )


def kernel(x, codebook, temperature):
    ids2, d2 = _dist_argmin(x, codebook.T)
    ids = ids2.reshape(N_TOK)
    emb = _sc_gather()(codebook, ids.reshape(_NW, _NCH, _CHUNK))
    loss = 1.25 * (jnp.sum(d2) / (N_TOK * D))
    return emb, ids, loss


# speedup vs baseline: 1.0776x; 1.0776x over previous
import functools

import jax
import jax.numpy as jnp
from jax import lax
from jax.experimental import pallas as pl
from jax.experimental.pallas import tpu as pltpu
from jax.experimental.pallas import tpu_sc as plsc

N_TOK = 8192
N_CB = 8192
D = 256

BM = 512
BW = 1024
NF = N_CB // BW
NI = N_TOK // BM


def _dist_argmin_body(x_ref, c_ref, ids_ref, d2_ref, ct_ref, c2_ref):
    i = pl.program_id(0)

    @pl.when(i == 0)
    def _cache_ct_c2():
        ct = c_ref[...].T
        ct_ref[...] = ct
        c2_ref[...] = jnp.sum(ct * ct, axis=0, keepdims=True)

    x = x_ref[...]
    xm2 = x * -2.0

    vals = [
        c2_ref[:, jj * BW:(jj + 1) * BW]
        + lax.dot_general(xm2, ct_ref[:, jj * BW:(jj + 1) * BW],
                          (((1,), (0,)), ((), ())),
                          preferred_element_type=jnp.float32)
        for jj in range(NF)
    ]
    idxs = [jnp.full((BM, BW), jj, jnp.int32) for jj in range(NF)]
    while len(vals) > 1:
        nv, ni_ = [], []
        for k in range(0, len(vals), 2):
            va, vb = vals[k], vals[k + 1]
            ia, ib = idxs[k], idxs[k + 1]
            take_b = vb < va
            nv.append(jnp.minimum(va, vb))
            ni_.append(jnp.where(take_b, ib, ia))
        vals, idxs = nv, ni_
    a, ja = vals[0], idxs[0]

    m = jnp.min(a, axis=1, keepdims=True)
    col = ja * BW + lax.broadcasted_iota(jnp.int32, (BM, BW), 1)
    ids_ref[...] = jnp.min(jnp.where(a == m, col, N_CB), axis=1,
                           keepdims=True)
    x2 = jnp.sum(x * x, axis=1, keepdims=True)
    d2_ref[...] = jnp.maximum(x2 + m, 0.0)


_dist_argmin = pl.pallas_call(
    _dist_argmin_body,
    grid=(NI,),
    in_specs=[
        pl.BlockSpec((BM, D), lambda i: (i, 0)),
        pl.BlockSpec((N_CB, D), lambda i: (0, 0)),
    ],
    out_specs=[
        pl.BlockSpec((BM, 1), lambda i: (i, 0)),
        pl.BlockSpec((BM, 1), lambda i: (i, 0)),
    ],
    out_shape=[
        jax.ShapeDtypeStruct((N_TOK, 1), jnp.int32),
        jax.ShapeDtypeStruct((N_TOK, 1), jnp.float32),
    ],
    scratch_shapes=[
        pltpu.VMEM((D, N_CB), jnp.float32),
        pltpu.VMEM((1, N_CB), jnp.float32),
    ],
    compiler_params=pltpu.CompilerParams(
        dimension_semantics=("arbitrary",),
    ),
)

_NC = 2
_NS = 16
_NW = _NC * _NS
_BPW = N_TOK // _NW
_CHUNK = 128
_NCH = _BPW // _CHUNK


def _sc_gather_body(table_hbm, idx_hbm, out_hbm, idx_v, rows_v, sem):
    wid = lax.axis_index("s") * _NC + lax.axis_index("c")
    base = wid * _BPW
    pltpu.sync_copy(idx_hbm.at[wid], idx_v)
    copies = []
    for k in range(_NCH):
        copies.append(pltpu.async_copy(
            table_hbm.at[idx_v.at[k]],
            rows_v.at[pl.ds(k * _CHUNK, _CHUNK)],
            sem,
        ))
    for cp in copies:
        cp.wait()
    pltpu.sync_copy(rows_v, out_hbm.at[pl.ds(base, _BPW)])


@functools.cache
def _sc_gather():
    return pl.kernel(
        _sc_gather_body,
        mesh=plsc.VectorSubcoreMesh(core_axis_name="c", subcore_axis_name="s"),
        out_type=jax.ShapeDtypeStruct((N_TOK, D), jnp.float32),
        scratch_types=[
            pltpu.VMEM((_NCH, _CHUNK), jnp.int32),
            pltpu.VMEM((_BPW, D), jnp.float32),
            pltpu.SemaphoreType.DMA,
        ],
    )


def kernel(x, codebook, temperature):
    ids2, d2 = _dist_argmin(x, codebook)
    ids = ids2.reshape(N_TOK)
    emb = _sc_gather()(codebook, ids.reshape(_NW, _NCH, _CHUNK))
    loss = 1.25 * (jnp.sum(d2) / (N_TOK * D))
    return emb, ids, loss


# speedup vs baseline: 1.1979x; 1.1117x over previous
import functools

import jax
import jax.numpy as jnp
from jax import lax
from jax.experimental import pallas as pl
from jax.experimental.pallas import tpu as pltpu
from jax.experimental.pallas import tpu_sc as plsc

N_TOK = 8192
N_CB = 8192
D = 256

BM = 512
BW = 1024
NF = N_CB // BW
NI = N_TOK // BM


def _dist_argmin_body(x_ref, c_ref, ids_ref, d2_ref, ct_ref, c2_ref):
    i = pl.program_id(0)

    @pl.when(i == 0)
    def _cache_ct_c2():
        ct = c_ref[...].T
        ct_ref[...] = ct
        c2_ref[...] = jnp.sum(ct * ct, axis=0, keepdims=True)

    x = x_ref[...]
    xm2 = x * -2.0

    def block_scores(jj):
        return (c2_ref[:, jj * BW:(jj + 1) * BW]
                + lax.dot_general(xm2, ct_ref[:, jj * BW:(jj + 1) * BW],
                                  (((1,), (0,)), ((), ())),
                                  preferred_element_type=jnp.float32))

    a = block_scores(0)
    ja = jnp.zeros((BM, BW), jnp.int32)
    for jj in range(1, NF):
        s = block_scores(jj)
        ja = jnp.where(s < a, jj, ja)
        a = jnp.minimum(a, s)

    m = jnp.min(a, axis=1, keepdims=True)
    col = ja * BW + lax.broadcasted_iota(jnp.int32, (BM, BW), 1)
    ids_ref[...] = jnp.min(jnp.where(a == m, col, N_CB), axis=1,
                           keepdims=True)
    x2 = jnp.sum(x * x, axis=1, keepdims=True)
    d2_ref[...] = jnp.maximum(x2 + m, 0.0)


_dist_argmin = pl.pallas_call(
    _dist_argmin_body,
    grid=(NI,),
    in_specs=[
        pl.BlockSpec((BM, D), lambda i: (i, 0)),
        pl.BlockSpec((N_CB, D), lambda i: (0, 0)),
    ],
    out_specs=[
        pl.BlockSpec((BM, 1), lambda i: (i, 0)),
        pl.BlockSpec((BM, 1), lambda i: (i, 0)),
    ],
    out_shape=[
        jax.ShapeDtypeStruct((N_TOK, 1), jnp.int32),
        jax.ShapeDtypeStruct((N_TOK, 1), jnp.float32),
    ],
    scratch_shapes=[
        pltpu.VMEM((D, N_CB), jnp.float32),
        pltpu.VMEM((1, N_CB), jnp.float32),
    ],
    compiler_params=pltpu.CompilerParams(
        dimension_semantics=("arbitrary",),
    ),
)

_NC = 2
_NS = 16
_NW = _NC * _NS
_BPW = N_TOK // _NW
_CHUNK = 128
_NCH = _BPW // _CHUNK


def _sc_gather_body(table_hbm, idx_hbm, out_hbm, idx_v, rows_v, sem):
    wid = lax.axis_index("s") * _NC + lax.axis_index("c")
    base = wid * _BPW
    pltpu.sync_copy(idx_hbm.at[wid], idx_v)
    copies = []
    for k in range(_NCH):
        copies.append(pltpu.async_copy(
            table_hbm.at[idx_v.at[k]],
            rows_v.at[pl.ds(k * _CHUNK, _CHUNK)],
            sem,
        ))
    for cp in copies:
        cp.wait()
    pltpu.sync_copy(rows_v, out_hbm.at[pl.ds(base, _BPW)])


@functools.cache
def _sc_gather():
    return pl.kernel(
        _sc_gather_body,
        mesh=plsc.VectorSubcoreMesh(core_axis_name="c", subcore_axis_name="s"),
        out_type=jax.ShapeDtypeStruct((N_TOK, D), jnp.float32),
        scratch_types=[
            pltpu.VMEM((_NCH, _CHUNK), jnp.int32),
            pltpu.VMEM((_BPW, D), jnp.float32),
            pltpu.SemaphoreType.DMA,
        ],
    )


def kernel(x, codebook, temperature):
    ids2, d2 = _dist_argmin(x, codebook)
    ids = ids2.reshape(N_TOK)
    emb = _sc_gather()(codebook, ids.reshape(_NW, _NCH, _CHUNK))
    loss = 1.25 * (jnp.sum(d2) / (N_TOK * D))
    return emb, ids, loss
